# baseline (speedup 1.0000x reference)
import functools

import jax
import jax.numpy as jnp
from jax import lax
from jax.experimental import pallas as pl
from jax.experimental.pallas import tpu as pltpu
from jax.experimental.pallas import tpu_sc as plsc

N = 10000
E = 320000
NPAD = 10240
H = 128
BOT = 32
NC = 2
NS = 16
NW = NC * NS
EPT = E // NW
EPP = 10240
K = 128
G = EPP // K
SEG = 8
NSEG = G // SEG
SINK = N
RPT = NPAD // NS
BNS = (1.0 + 1e-5) ** -0.5
PREC = lax.Precision.HIGHEST

_mesh = plsc.VectorSubcoreMesh(
    core_axis_name="c", subcore_axis_name="s", num_cores=NC, num_subcores=NS
)


def _deg_body(dst_hbm, hist_hbm, dstv, hist):
    c = lax.axis_index("c")
    s = lax.axis_index("s")
    wid = c * NS + s

    def zero_body(i, _):
        hist[pl.ds(i * 16, 16)] = jnp.zeros((16,), jnp.float32)
        return 0

    lax.fori_loop(0, NPAD // 16, zero_body, 0)
    pltpu.sync_copy(dst_hbm.at[wid], dstv)
    ones16 = jnp.ones((16,), jnp.float32)

    def count_body(i, _):
        idx16 = dstv[pl.ds(i * 16, 16)]
        plsc.addupdate_scatter(hist, [idx16], ones16)
        return 0

    lax.fori_loop(0, EPT // 16, count_body, 0)
    pltpu.sync_copy(hist, hist_hbm.at[wid])


_deg_kernel = pl.kernel(
    _deg_body,
    out_type=jax.ShapeDtypeStruct((NW, NPAD), jnp.float32),
    mesh=_mesh,
    compiler_params=pltpu.CompilerParams(needs_layout_passes=False),
    scratch_types=[
        pltpu.VMEM((EPT,), jnp.int32),
        pltpu.VMEM((NPAD,), jnp.float32),
    ],
)


def _spmm_body(table_hbm, src_hbm, dst_hbm, acc_hbm, srcv, didx, rows, accsp, sem, isem):
    c = lax.axis_index("c")
    s = lax.axis_index("s")
    wid = c * NS + s

    def zrow(i, _):
        r = i // 8
        k = i % 8
        rows[0, r, pl.ds(k * 16, 16)] = jnp.zeros((16,), jnp.float32)
        return 0

    lax.fori_loop(0, K * 8, zrow, 0)
    for b in range(RPT // K):
        pltpu.sync_copy(rows.at[0], accsp.at[pl.ds(s * RPT + b * K, K)])
    plsc.subcore_barrier()

    cp_s = pltpu.async_copy(src_hbm.at[wid], srcv, isem)
    pltpu.sync_copy(dst_hbm.at[wid, pl.ds(0, SEG)], didx.at[0])
    cp_s.wait()

    pltpu.async_copy(table_hbm.at[srcv.at[pl.ds(0, K)]], rows.at[0], sem.at[0])

    def seg_body(sg, _):
        sb = sg % 2

        @pl.when(sg + 1 < NSEG)
        def _():
            pltpu.async_copy(
                dst_hbm.at[wid, pl.ds((sg + 1) * SEG, SEG)], didx.at[1 - sb], isem
            )

        for j in range(SEG):
            g = sg * SEG + j
            b = j % 2

            @pl.when(g + 1 < G)
            def _():
                pltpu.async_copy(
                    table_hbm.at[srcv.at[pl.ds((g + 1) * K, K)]],
                    rows.at[1 - b],
                    sem.at[1 - b],
                )

            pltpu.make_async_copy(
                table_hbm.at[srcv.at[pl.ds(g * K, K)]], rows.at[b], sem.at[b]
            ).wait()
            pltpu.sync_copy(rows.at[b], accsp.at[didx.at[sb, j]], add=True)

        @pl.when(sg + 1 < NSEG)
        def _():
            pltpu.make_async_copy(
                dst_hbm.at[wid, pl.ds((sg + 1) * SEG, SEG)], didx.at[1 - sb], isem
            ).wait()

        return 0

    lax.fori_loop(0, NSEG, seg_body, 0)
    plsc.subcore_barrier()
    for b in range(RPT // K):
        pltpu.sync_copy(
            accsp.at[pl.ds(s * RPT + b * K, K)],
            acc_hbm.at[c, pl.ds(s * RPT + b * K, K)],
        )


_spmm_kernel = pl.kernel(
    _spmm_body,
    out_type=jax.ShapeDtypeStruct((NC, NPAD, H), jnp.float32),
    mesh=_mesh,
    compiler_params=pltpu.CompilerParams(needs_layout_passes=False),
    scratch_types=[
        pltpu.VMEM((EPP,), jnp.int32),
        pltpu.VMEM((2, SEG, K), jnp.int32),
        pltpu.VMEM((2, K, H), jnp.float32),
        pltpu.VMEM_SHARED((NPAD, H), jnp.float32),
        pltpu.SemaphoreType.DMA((2,)),
        pltpu.SemaphoreType.DMA,
    ],
)


def _pre_body(x_ref, hist_ref, w1_ref, h1p_ref, dis_ref):
    ones = jnp.ones((NW, 1), jnp.float32)
    deg = 1.0 + lax.dot_general(
        hist_ref[...], ones, (((0,), (0,)), ((), ())), precision=PREC
    )
    dis = lax.rsqrt(deg)
    dis_ref[...] = dis
    h1p_ref[...] = jnp.dot(x_ref[...], w1_ref[...], precision=PREC) * dis


def _pre_stage(x_pad, hist, w1):
    blk = 1024
    grid = NPAD // blk
    return pl.pallas_call(
        _pre_body,
        grid=(grid,),
        in_specs=[
            pl.BlockSpec((blk, H), lambda i: (i, 0)),
            pl.BlockSpec((NW, blk), lambda i: (0, i)),
            pl.BlockSpec((H, H), lambda i: (0, 0)),
        ],
        out_specs=[
            pl.BlockSpec((blk, H), lambda i: (i, 0)),
            pl.BlockSpec((blk, 1), lambda i: (i, 0)),
        ],
        out_shape=[
            jax.ShapeDtypeStruct((NPAD, H), jnp.float32),
            jax.ShapeDtypeStruct((NPAD, 1), jnp.float32),
        ],
    )(x_pad, hist, w1)


def _mid_body(acc_ref, h1p_ref, dis_ref, b1_ref, g1_ref, be1_ref, w2_ref, h2p_ref):
    dis = dis_ref[...]
    g = (acc_ref[0, :, :] + acc_ref[1, :, :] + h1p_ref[...]) * dis
    pre = (g + b1_ref[...]) * BNS * g1_ref[...] + be1_ref[...]
    out1 = jnp.maximum(pre, 0.0)
    h2p_ref[...] = jnp.dot(out1, w2_ref[...], precision=PREC) * dis


def _mid_stage(acc, h1p, dis, b1, gamma1, beta1, w2):
    blk = 1024
    grid = NPAD // blk
    return pl.pallas_call(
        _mid_body,
        grid=(grid,),
        in_specs=[
            pl.BlockSpec((NC, blk, H), lambda i: (0, i, 0)),
            pl.BlockSpec((blk, H), lambda i: (i, 0)),
            pl.BlockSpec((blk, 1), lambda i: (i, 0)),
            pl.BlockSpec((H,), lambda i: (0,)),
            pl.BlockSpec((H,), lambda i: (0,)),
            pl.BlockSpec((H,), lambda i: (0,)),
            pl.BlockSpec((H, H), lambda i: (0, 0)),
        ],
        out_specs=pl.BlockSpec((blk, H), lambda i: (i, 0)),
        out_shape=jax.ShapeDtypeStruct((NPAD, H), jnp.float32),
    )(acc, h1p, dis, b1, gamma1, beta1, w2)


_TBLK = 1000
_TCELLS = _TBLK // 100
_TGRID = N // _TBLK + 1


def _tail_body(acc_ref, h2p_ref, dis_ref, b2_ref, g2_ref, be2_ref,
               aw1_ref, ab1_ref, aw2_ref, we_ref, bee_ref, ge_ref, bte_ref,
               wb_ref, bb_ref, gb_ref, btb_ref, emb_ref, out_ref, pooled_s):
    i = pl.program_id(0)
    g = (acc_ref[0, :, :] + acc_ref[1, :, :] + h2p_ref[...]) * dis_ref[...]
    pre = (g + b2_ref[...]) * BNS * g2_ref[...] + be2_ref[...]
    emb = jnp.maximum(pre, 0.0)
    emb_ref[...] = emb
    t = jnp.tanh(jnp.dot(emb, aw1_ref[...], precision=PREC) + ab1_ref[...])
    sc = jnp.dot(t, aw2_ref[...], precision=PREC)
    ex = jnp.exp(sc)
    for j in range(_TCELLS):
        ej = ex[j * 100:(j + 1) * 100]
        xj = emb[j * 100:(j + 1) * 100]
        w = ej / jnp.sum(ej, axis=0, keepdims=True)
        pooled_s[pl.ds(i * _TCELLS + j, 1), :] = (
            jnp.sum(xj * w, axis=0).reshape(1, H))

    @pl.when(i == _TGRID - 1)
    def _():
        p = pooled_s[pl.ds(0, 100), :]
        e1 = jnp.dot(p, we_ref[...], precision=PREC) + bee_ref[...]
        e1 = jnp.maximum(e1 * BNS * ge_ref[...] + bte_ref[...], 0.0)
        o = jnp.dot(e1, wb_ref[...], precision=PREC) + bb_ref[...]
        out_ref[...] = jnp.maximum(o * BNS * gb_ref[...] + btb_ref[...], 0.0)


def _tail_stage(acc, h2p, dis, b2, gamma2, beta2, aw1, ab1, aw2,
                we, be, gammae, betae, wb, bb, gammab, betab):
    last = N // _TBLK - 1
    clamp = lambda i: jnp.minimum(i, last)
    return pl.pallas_call(
        _tail_body,
        grid=(_TGRID,),
        in_specs=[
            pl.BlockSpec((NC, _TBLK, H), lambda i: (0, clamp(i), 0)),
            pl.BlockSpec((_TBLK, H), lambda i: (clamp(i), 0)),
            pl.BlockSpec((_TBLK, 1), lambda i: (clamp(i), 0)),
            pl.BlockSpec((H,), lambda i: (0,)),
            pl.BlockSpec((H,), lambda i: (0,)),
            pl.BlockSpec((H,), lambda i: (0,)),
            pl.BlockSpec((H, H // 2), lambda i: (0, 0)),
            pl.BlockSpec((H // 2,), lambda i: (0,)),
            pl.BlockSpec((H // 2, 1), lambda i: (0, 0)),
            pl.BlockSpec((H, H), lambda i: (0, 0)),
            pl.BlockSpec((H,), lambda i: (0,)),
            pl.BlockSpec((H,), lambda i: (0,)),
            pl.BlockSpec((H,), lambda i: (0,)),
            pl.BlockSpec((H, BOT), lambda i: (0, 0)),
            pl.BlockSpec((BOT,), lambda i: (0,)),
            pl.BlockSpec((BOT,), lambda i: (0,)),
            pl.BlockSpec((BOT,), lambda i: (0,)),
        ],
        out_specs=[
            pl.BlockSpec((_TBLK, H), lambda i: (clamp(i), 0)),
            pl.BlockSpec((100, BOT), lambda i: (0, 0)),
        ],
        out_shape=[
            jax.ShapeDtypeStruct((N, H), jnp.float32),
            jax.ShapeDtypeStruct((100, BOT), jnp.float32),
        ],
        scratch_shapes=[pltpu.VMEM((104, H), jnp.float32)],
    )(acc, h2p, dis, b2, gamma2, beta2, aw1, ab1, aw2,
      we, be, gammae, betae, wb, bb, gammab, betab)


@jax.jit
def kernel(x, edge_index, W1, b1, W2, b2, gamma1, beta1, gamma2, beta2,
           Aw1, Ab1, Aw2, Ab2, We, be, gammae, betae, Wb, bb, gammab, betab):
    pad = NW * EPP - E
    pad_src = jnp.arange(pad, dtype=edge_index.dtype) % N
    src_r = jnp.concatenate([edge_index[0], pad_src]).reshape(NW, EPP)
    sinks = SINK + jnp.arange(pad, dtype=edge_index.dtype) % (NPAD - N)
    dst_r = jnp.concatenate([edge_index[1], sinks]).reshape(NW, G, K)
    dst_flat = edge_index[1].reshape(NW, EPT)
    x_pad = jnp.pad(x, ((0, NPAD - N), (0, 0)))

    hist = _deg_kernel(dst_flat)
    h1p, dis = _pre_stage(x_pad, hist, W1)
    acc1 = _spmm_kernel(h1p, src_r, dst_r)
    h2p = _mid_stage(acc1, h1p, dis, b1, gamma1, beta1, W2)
    acc2 = _spmm_kernel(h2p, src_r, dst_r)
    emb, out = _tail_stage(acc2, h2p, dis, b2, gamma2, beta2, Aw1, Ab1, Aw2,
                           We, be, gammae, betae, Wb, bb, gammab, betab)
    return (out, emb)


# speedup vs baseline: 1.0249x; 1.0249x over previous
import jax
import jax.numpy as jnp
from jax import lax
from jax.experimental import pallas as pl
from jax.experimental.pallas import tpu as pltpu
from jax.experimental.pallas import tpu_sc as plsc

N = 10000
E = 320000
NPAD = 10240
H = 128
BOT = 32
NC = 2
NS = 16
NW = NC * NS
EPT = E // NW
EPP = 10240
K = 128
G = EPP // K
SEG = 8
NSEG = G // SEG
SINK = N
RPT = NPAD // NS
BNS = (1.0 + 1e-5) ** -0.5
PREC = lax.Precision.HIGHEST

_mesh = plsc.VectorSubcoreMesh(
    core_axis_name="c", subcore_axis_name="s", num_cores=NC, num_subcores=NS
)


def _deg_body(dst_hbm, hist_hbm, dstv, hist):
    c = lax.axis_index("c")
    s = lax.axis_index("s")
    wid = c * NS + s

    def zero_body(i, _):
        hist[pl.ds(i * 16, 16)] = jnp.zeros((16,), jnp.float32)
        return 0

    lax.fori_loop(0, NPAD // 16, zero_body, 0)
    pltpu.sync_copy(dst_hbm.at[wid], dstv)
    ones16 = jnp.ones((16,), jnp.float32)

    def count_body(i, _):
        idx16 = dstv[pl.ds(i * 16, 16)]
        plsc.addupdate_scatter(hist, [idx16], ones16)
        return 0

    lax.fori_loop(0, EPT // 16, count_body, 0)
    pltpu.sync_copy(hist, hist_hbm.at[wid])


_deg_kernel = pl.kernel(
    _deg_body,
    out_type=jax.ShapeDtypeStruct((NW, NPAD), jnp.float32),
    mesh=_mesh,
    compiler_params=pltpu.CompilerParams(needs_layout_passes=False),
    scratch_types=[
        pltpu.VMEM((EPT,), jnp.int32),
        pltpu.VMEM((NPAD,), jnp.float32),
    ],
)


def _spmm_body(table_hbm, src_hbm, dst_hbm, acc_hbm, srcv, didx, rows, accsp, sem, isem):
    c = lax.axis_index("c")
    s = lax.axis_index("s")
    wid = c * NS + s

    def zrow(i, _):
        r = i // 8
        k = i % 8
        rows[0, r, pl.ds(k * 16, 16)] = jnp.zeros((16,), jnp.float32)
        return 0

    lax.fori_loop(0, K * 8, zrow, 0)
    for b in range(RPT // K):
        pltpu.sync_copy(rows.at[0], accsp.at[pl.ds(s * RPT + b * K, K)])
    plsc.subcore_barrier()

    cp_s = pltpu.async_copy(src_hbm.at[wid], srcv, isem)
    pltpu.sync_copy(dst_hbm.at[wid, pl.ds(0, SEG)], didx.at[0])
    cp_s.wait()

    pltpu.async_copy(table_hbm.at[srcv.at[pl.ds(0, K)]], rows.at[0], sem.at[0])

    def seg_body(sg, _):
        sb = sg % 2

        @pl.when(sg + 1 < NSEG)
        def _():
            pltpu.async_copy(
                dst_hbm.at[wid, pl.ds((sg + 1) * SEG, SEG)], didx.at[1 - sb], isem
            )

        for j in range(SEG):
            g = sg * SEG + j
            b = j % 2

            @pl.when(g + 1 < G)
            def _():
                pltpu.async_copy(
                    table_hbm.at[srcv.at[pl.ds((g + 1) * K, K)]],
                    rows.at[1 - b],
                    sem.at[1 - b],
                )

            pltpu.make_async_copy(
                table_hbm.at[srcv.at[pl.ds(g * K, K)]], rows.at[b], sem.at[b]
            ).wait()
            pltpu.sync_copy(rows.at[b], accsp.at[didx.at[sb, j]], add=True)

        @pl.when(sg + 1 < NSEG)
        def _():
            pltpu.make_async_copy(
                dst_hbm.at[wid, pl.ds((sg + 1) * SEG, SEG)], didx.at[1 - sb], isem
            ).wait()

        return 0

    lax.fori_loop(0, NSEG, seg_body, 0)
    plsc.subcore_barrier()
    for b in range(RPT // K):
        pltpu.sync_copy(
            accsp.at[pl.ds(s * RPT + b * K, K)],
            acc_hbm.at[c, pl.ds(s * RPT + b * K, K)],
        )


_spmm_kernel = pl.kernel(
    _spmm_body,
    out_type=jax.ShapeDtypeStruct((NC, NPAD, H), jnp.float32),
    mesh=_mesh,
    compiler_params=pltpu.CompilerParams(needs_layout_passes=False),
    scratch_types=[
        pltpu.VMEM((EPP,), jnp.int32),
        pltpu.VMEM((2, SEG, K), jnp.int32),
        pltpu.VMEM((2, K, H), jnp.float32),
        pltpu.VMEM_SHARED((NPAD, H), jnp.float32),
        pltpu.SemaphoreType.DMA((2,)),
        pltpu.SemaphoreType.DMA,
    ],
)


def _pre_body(x_ref, hist_ref, w1_ref, h1p_ref, dis_ref):
    ones = jnp.ones((NW, 1), jnp.float32)
    deg = 1.0 + lax.dot_general(
        hist_ref[...], ones, (((0,), (0,)), ((), ())), precision=PREC
    )
    dis = lax.rsqrt(deg)
    dis_ref[...] = dis
    h1p_ref[...] = jnp.dot(x_ref[...], w1_ref[...], precision=PREC) * dis


def _pre_stage(x_pad, hist, w1):
    blk = 1024
    grid = NPAD // blk
    return pl.pallas_call(
        _pre_body,
        grid=(grid,),
        in_specs=[
            pl.BlockSpec((blk, H), lambda i: (i, 0)),
            pl.BlockSpec((NW, blk), lambda i: (0, i)),
            pl.BlockSpec((H, H), lambda i: (0, 0)),
        ],
        out_specs=[
            pl.BlockSpec((blk, H), lambda i: (i, 0)),
            pl.BlockSpec((blk, 1), lambda i: (i, 0)),
        ],
        out_shape=[
            jax.ShapeDtypeStruct((NPAD, H), jnp.float32),
            jax.ShapeDtypeStruct((NPAD, 1), jnp.float32),
        ],
    )(x_pad, hist, w1)


def _mid_body(acc_ref, h1p_ref, dis_ref, b1_ref, g1_ref, be1_ref, w2_ref, h2p_ref):
    dis = dis_ref[...]
    g = (acc_ref[0, :, :] + acc_ref[1, :, :] + h1p_ref[...]) * dis
    pre = (g + b1_ref[...]) * BNS * g1_ref[...] + be1_ref[...]
    out1 = jnp.maximum(pre, 0.0)
    h2p_ref[...] = jnp.dot(out1, w2_ref[...], precision=PREC) * dis


def _mid_stage(acc, h1p, dis, b1, gamma1, beta1, w2):
    blk = 1024
    grid = NPAD // blk
    return pl.pallas_call(
        _mid_body,
        grid=(grid,),
        in_specs=[
            pl.BlockSpec((NC, blk, H), lambda i: (0, i, 0)),
            pl.BlockSpec((blk, H), lambda i: (i, 0)),
            pl.BlockSpec((blk, 1), lambda i: (i, 0)),
            pl.BlockSpec((H,), lambda i: (0,)),
            pl.BlockSpec((H,), lambda i: (0,)),
            pl.BlockSpec((H,), lambda i: (0,)),
            pl.BlockSpec((H, H), lambda i: (0, 0)),
        ],
        out_specs=pl.BlockSpec((blk, H), lambda i: (i, 0)),
        out_shape=jax.ShapeDtypeStruct((NPAD, H), jnp.float32),
    )(acc, h1p, dis, b1, gamma1, beta1, w2)


_TBLK = 400
_TCELLS = _TBLK // 100
_TGRID = N // _TBLK + 1


def _tail_body(acc_ref, h2p_ref, dis_ref, b2_ref, g2_ref, be2_ref,
               aw1_ref, ab1_ref, aw2_ref, we_ref, bee_ref, ge_ref, bte_ref,
               wb_ref, bb_ref, gb_ref, btb_ref, emb_ref, out_ref, pooled_s):
    i = pl.program_id(0)
    g = (acc_ref[0, :, :] + acc_ref[1, :, :] + h2p_ref[...]) * dis_ref[...]
    pre = (g + b2_ref[...]) * BNS * g2_ref[...] + be2_ref[...]
    emb = jnp.maximum(pre, 0.0)
    emb_ref[...] = emb
    t = jnp.tanh(jnp.dot(emb, aw1_ref[...], precision=PREC) + ab1_ref[...])
    sc = jnp.dot(t, aw2_ref[...], precision=PREC)
    ex = jnp.exp(sc)
    for j in range(_TCELLS):
        ej = ex[j * 100:(j + 1) * 100]
        xj = emb[j * 100:(j + 1) * 100]
        w = ej / jnp.sum(ej, axis=0, keepdims=True)
        pooled_s[pl.ds(i * _TCELLS + j, 1), :] = (
            jnp.sum(xj * w, axis=0).reshape(1, H))

    @pl.when(i == _TGRID - 1)
    def _():
        p = pooled_s[pl.ds(0, 100), :]
        e1 = jnp.dot(p, we_ref[...], precision=PREC) + bee_ref[...]
        e1 = jnp.maximum(e1 * BNS * ge_ref[...] + bte_ref[...], 0.0)
        o = jnp.dot(e1, wb_ref[...], precision=PREC) + bb_ref[...]
        out_ref[...] = jnp.maximum(o * BNS * gb_ref[...] + btb_ref[...], 0.0)


def _tail_stage(acc, h2p, dis, b2, gamma2, beta2, aw1, ab1, aw2,
                we, be, gammae, betae, wb, bb, gammab, betab):
    last = N // _TBLK - 1
    clamp = lambda i: jnp.minimum(i, last)
    return pl.pallas_call(
        _tail_body,
        grid=(_TGRID,),
        in_specs=[
            pl.BlockSpec((NC, _TBLK, H), lambda i: (0, clamp(i), 0)),
            pl.BlockSpec((_TBLK, H), lambda i: (clamp(i), 0)),
            pl.BlockSpec((_TBLK, 1), lambda i: (clamp(i), 0)),
            pl.BlockSpec((H,), lambda i: (0,)),
            pl.BlockSpec((H,), lambda i: (0,)),
            pl.BlockSpec((H,), lambda i: (0,)),
            pl.BlockSpec((H, H // 2), lambda i: (0, 0)),
            pl.BlockSpec((H // 2,), lambda i: (0,)),
            pl.BlockSpec((H // 2, 1), lambda i: (0, 0)),
            pl.BlockSpec((H, H), lambda i: (0, 0)),
            pl.BlockSpec((H,), lambda i: (0,)),
            pl.BlockSpec((H,), lambda i: (0,)),
            pl.BlockSpec((H,), lambda i: (0,)),
            pl.BlockSpec((H, BOT), lambda i: (0, 0)),
            pl.BlockSpec((BOT,), lambda i: (0,)),
            pl.BlockSpec((BOT,), lambda i: (0,)),
            pl.BlockSpec((BOT,), lambda i: (0,)),
        ],
        out_specs=[
            pl.BlockSpec((_TBLK, H), lambda i: (clamp(i), 0)),
            pl.BlockSpec((100, BOT), lambda i: (0, 0)),
        ],
        out_shape=[
            jax.ShapeDtypeStruct((N, H), jnp.float32),
            jax.ShapeDtypeStruct((100, BOT), jnp.float32),
        ],
        scratch_shapes=[pltpu.VMEM((104, H), jnp.float32)],
    )(acc, h2p, dis, b2, gamma2, beta2, aw1, ab1, aw2,
      we, be, gammae, betae, wb, bb, gammab, betab)


@jax.jit
def kernel(x, edge_index, W1, b1, W2, b2, gamma1, beta1, gamma2, beta2,
           Aw1, Ab1, Aw2, Ab2, We, be, gammae, betae, Wb, bb, gammab, betab):
    pad = NW * EPP - E
    pad_src = jnp.arange(pad, dtype=edge_index.dtype) % N
    src_r = jnp.concatenate([edge_index[0], pad_src]).reshape(NW, EPP)
    sinks = SINK + jnp.arange(pad, dtype=edge_index.dtype) % (NPAD - N)
    dst_r = jnp.concatenate([edge_index[1], sinks]).reshape(NW, G, K)
    dst_flat = edge_index[1].reshape(NW, EPT)
    x_pad = jnp.pad(x, ((0, NPAD - N), (0, 0)))

    hist = _deg_kernel(dst_flat)
    h1p, dis = _pre_stage(x_pad, hist, W1)
    acc1 = _spmm_kernel(h1p, src_r, dst_r)
    h2p = _mid_stage(acc1, h1p, dis, b1, gamma1, beta1, W2)
    acc2 = _spmm_kernel(h2p, src_r, dst_r)
    emb, out = _tail_stage(acc2, h2p, dis, b2, gamma2, beta2, Aw1, Ab1, Aw2,
                           We, be, gammae, betae, Wb, bb, gammab, betab)
    return (out, emb)


# speedup vs baseline: 1.0433x; 1.0180x over previous
import jax
import jax.numpy as jnp
from jax import lax
from jax.experimental import pallas as pl
from jax.experimental.pallas import tpu as pltpu
from jax.experimental.pallas import tpu_sc as plsc

N = 10000
E = 320000
NPAD = 10240
H = 128
BOT = 32
NC = 2
NS = 16
NW = NC * NS
EPT = E // NW
EPP = 10240
K = 128
G = EPP // K
SEG = 8
NSEG = G // SEG
SINK = N
RPT = NPAD // NS
BNS = (1.0 + 1e-5) ** -0.5
PREC = lax.Precision.HIGHEST

_mesh = plsc.VectorSubcoreMesh(
    core_axis_name="c", subcore_axis_name="s", num_cores=NC, num_subcores=NS
)


def _deg_body(dst_hbm, hist_hbm, dstv, hist):
    c = lax.axis_index("c")
    s = lax.axis_index("s")
    wid = c * NS + s

    def zero_body(i, _):
        hist[pl.ds(i * 16, 16)] = jnp.zeros((16,), jnp.float32)
        return 0

    lax.fori_loop(0, NPAD // 16, zero_body, 0)
    pltpu.sync_copy(dst_hbm.at[wid], dstv)
    ones16 = jnp.ones((16,), jnp.float32)

    def count_body(i, _):
        idx16 = dstv[pl.ds(i * 16, 16)]
        plsc.addupdate_scatter(hist, [idx16], ones16)
        return 0

    lax.fori_loop(0, EPT // 16, count_body, 0)
    pltpu.sync_copy(hist, hist_hbm.at[wid])


_deg_kernel = pl.kernel(
    _deg_body,
    out_type=jax.ShapeDtypeStruct((NW, NPAD), jnp.float32),
    mesh=_mesh,
    compiler_params=pltpu.CompilerParams(needs_layout_passes=False),
    scratch_types=[
        pltpu.VMEM((EPT,), jnp.int32),
        pltpu.VMEM((NPAD,), jnp.float32),
    ],
)


def _spmm_body(table_hbm, src_hbm, dst_hbm, acc_hbm, srcv, didx, rows, accsp, sem, isem):
    c = lax.axis_index("c")
    s = lax.axis_index("s")
    wid = c * NS + s

    def zrow(i, _):
        r = i // 8
        k = i % 8
        rows[0, r, pl.ds(k * 16, 16)] = jnp.zeros((16,), jnp.float32)
        return 0

    lax.fori_loop(0, K * 8, zrow, 0)
    for b in range(RPT // K):
        pltpu.sync_copy(rows.at[0], accsp.at[pl.ds(s * RPT + b * K, K)])
    plsc.subcore_barrier()

    cp_s = pltpu.async_copy(src_hbm.at[wid], srcv, isem)
    pltpu.sync_copy(dst_hbm.at[wid, pl.ds(0, SEG)], didx.at[0])
    cp_s.wait()

    pltpu.async_copy(table_hbm.at[srcv.at[pl.ds(0, K)]], rows.at[0], sem.at[0])

    def seg_body(sg, _):
        sb = sg % 2

        @pl.when(sg + 1 < NSEG)
        def _():
            pltpu.async_copy(
                dst_hbm.at[wid, pl.ds((sg + 1) * SEG, SEG)], didx.at[1 - sb], isem
            )

        for j in range(SEG):
            g = sg * SEG + j
            b = j % 2

            @pl.when(g + 1 < G)
            def _():
                pltpu.async_copy(
                    table_hbm.at[srcv.at[pl.ds((g + 1) * K, K)]],
                    rows.at[1 - b],
                    sem.at[1 - b],
                )

            pltpu.make_async_copy(
                table_hbm.at[srcv.at[pl.ds(g * K, K)]], rows.at[b], sem.at[b]
            ).wait()
            pltpu.sync_copy(rows.at[b], accsp.at[didx.at[sb, j]], add=True)

        @pl.when(sg + 1 < NSEG)
        def _():
            pltpu.make_async_copy(
                dst_hbm.at[wid, pl.ds((sg + 1) * SEG, SEG)], didx.at[1 - sb], isem
            ).wait()

        return 0

    lax.fori_loop(0, NSEG, seg_body, 0)
    plsc.subcore_barrier()
    for b in range(RPT // K):
        pltpu.sync_copy(
            accsp.at[pl.ds(s * RPT + b * K, K)],
            acc_hbm.at[c, pl.ds(s * RPT + b * K, K)],
        )


_spmm_kernel = pl.kernel(
    _spmm_body,
    out_type=jax.ShapeDtypeStruct((NC, NPAD, H), jnp.float32),
    mesh=_mesh,
    compiler_params=pltpu.CompilerParams(needs_layout_passes=False),
    scratch_types=[
        pltpu.VMEM((EPP,), jnp.int32),
        pltpu.VMEM((2, SEG, K), jnp.int32),
        pltpu.VMEM((2, K, H), jnp.float32),
        pltpu.VMEM_SHARED((NPAD, H), jnp.float32),
        pltpu.SemaphoreType.DMA((2,)),
        pltpu.SemaphoreType.DMA,
    ],
)


def _pre_body(x_ref, hist_ref, w1_ref, h1p_ref, dis_ref):
    ones = jnp.ones((NW, 1), jnp.float32)
    deg = 1.0 + lax.dot_general(
        hist_ref[...], ones, (((0,), (0,)), ((), ())), precision=PREC
    )
    dis = lax.rsqrt(deg)
    dis_ref[...] = dis
    h1p_ref[...] = jnp.dot(x_ref[...], w1_ref[...], precision=PREC) * dis


def _pre_stage(x_pad, hist, w1):
    blk = 2048
    grid = NPAD // blk
    return pl.pallas_call(
        _pre_body,
        grid=(grid,),
        in_specs=[
            pl.BlockSpec((blk, H), lambda i: (i, 0)),
            pl.BlockSpec((NW, blk), lambda i: (0, i)),
            pl.BlockSpec((H, H), lambda i: (0, 0)),
        ],
        out_specs=[
            pl.BlockSpec((blk, H), lambda i: (i, 0)),
            pl.BlockSpec((blk, 1), lambda i: (i, 0)),
        ],
        out_shape=[
            jax.ShapeDtypeStruct((NPAD, H), jnp.float32),
            jax.ShapeDtypeStruct((NPAD, 1), jnp.float32),
        ],
    )(x_pad, hist, w1)


def _mid_body(acc_ref, h1p_ref, dis_ref, b1_ref, g1_ref, be1_ref, w2_ref, h2p_ref):
    dis = dis_ref[...]
    g = (acc_ref[0, :, :] + acc_ref[1, :, :] + h1p_ref[...]) * dis
    pre = (g + b1_ref[...]) * BNS * g1_ref[...] + be1_ref[...]
    out1 = jnp.maximum(pre, 0.0)
    h2p_ref[...] = jnp.dot(out1, w2_ref[...], precision=PREC) * dis


def _mid_stage(acc, h1p, dis, b1, gamma1, beta1, w2):
    blk = 2048
    grid = NPAD // blk
    return pl.pallas_call(
        _mid_body,
        grid=(grid,),
        in_specs=[
            pl.BlockSpec((NC, blk, H), lambda i: (0, i, 0)),
            pl.BlockSpec((blk, H), lambda i: (i, 0)),
            pl.BlockSpec((blk, 1), lambda i: (i, 0)),
            pl.BlockSpec((H,), lambda i: (0,)),
            pl.BlockSpec((H,), lambda i: (0,)),
            pl.BlockSpec((H,), lambda i: (0,)),
            pl.BlockSpec((H, H), lambda i: (0, 0)),
        ],
        out_specs=pl.BlockSpec((blk, H), lambda i: (i, 0)),
        out_shape=jax.ShapeDtypeStruct((NPAD, H), jnp.float32),
    )(acc, h1p, dis, b1, gamma1, beta1, w2)


_TBLK = 400
_TCELLS = _TBLK // 100
_TGRID = N // _TBLK + 1


def _tail_body(acc_ref, h2p_ref, dis_ref, b2_ref, g2_ref, be2_ref,
               aw1_ref, ab1_ref, aw2_ref, we_ref, bee_ref, ge_ref, bte_ref,
               wb_ref, bb_ref, gb_ref, btb_ref, emb_ref, out_ref, pooled_s):
    i = pl.program_id(0)
    g = (acc_ref[0, :, :] + acc_ref[1, :, :] + h2p_ref[...]) * dis_ref[...]
    pre = (g + b2_ref[...]) * BNS * g2_ref[...] + be2_ref[...]
    emb = jnp.maximum(pre, 0.0)
    emb_ref[...] = emb
    t = jnp.tanh(jnp.dot(emb, aw1_ref[...], precision=PREC) + ab1_ref[...])
    sc = jnp.dot(t, aw2_ref[...], precision=PREC)
    ex = jnp.exp(sc)
    for j in range(_TCELLS):
        ej = ex[j * 100:(j + 1) * 100]
        xj = emb[j * 100:(j + 1) * 100]
        w = ej / jnp.sum(ej, axis=0, keepdims=True)
        pooled_s[pl.ds(i * _TCELLS + j, 1), :] = (
            jnp.sum(xj * w, axis=0).reshape(1, H))

    @pl.when(i == _TGRID - 1)
    def _():
        p = pooled_s[pl.ds(0, 100), :]
        e1 = jnp.dot(p, we_ref[...], precision=PREC) + bee_ref[...]
        e1 = jnp.maximum(e1 * BNS * ge_ref[...] + bte_ref[...], 0.0)
        o = jnp.dot(e1, wb_ref[...], precision=PREC) + bb_ref[...]
        out_ref[...] = jnp.maximum(o * BNS * gb_ref[...] + btb_ref[...], 0.0)


def _tail_stage(acc, h2p, dis, b2, gamma2, beta2, aw1, ab1, aw2,
                we, be, gammae, betae, wb, bb, gammab, betab):
    last = N // _TBLK - 1
    clamp = lambda i: jnp.minimum(i, last)
    return pl.pallas_call(
        _tail_body,
        grid=(_TGRID,),
        in_specs=[
            pl.BlockSpec((NC, _TBLK, H), lambda i: (0, clamp(i), 0)),
            pl.BlockSpec((_TBLK, H), lambda i: (clamp(i), 0)),
            pl.BlockSpec((_TBLK, 1), lambda i: (clamp(i), 0)),
            pl.BlockSpec((H,), lambda i: (0,)),
            pl.BlockSpec((H,), lambda i: (0,)),
            pl.BlockSpec((H,), lambda i: (0,)),
            pl.BlockSpec((H, H // 2), lambda i: (0, 0)),
            pl.BlockSpec((H // 2,), lambda i: (0,)),
            pl.BlockSpec((H // 2, 1), lambda i: (0, 0)),
            pl.BlockSpec((H, H), lambda i: (0, 0)),
            pl.BlockSpec((H,), lambda i: (0,)),
            pl.BlockSpec((H,), lambda i: (0,)),
            pl.BlockSpec((H,), lambda i: (0,)),
            pl.BlockSpec((H, BOT), lambda i: (0, 0)),
            pl.BlockSpec((BOT,), lambda i: (0,)),
            pl.BlockSpec((BOT,), lambda i: (0,)),
            pl.BlockSpec((BOT,), lambda i: (0,)),
        ],
        out_specs=[
            pl.BlockSpec((_TBLK, H), lambda i: (clamp(i), 0)),
            pl.BlockSpec((100, BOT), lambda i: (0, 0)),
        ],
        out_shape=[
            jax.ShapeDtypeStruct((N, H), jnp.float32),
            jax.ShapeDtypeStruct((100, BOT), jnp.float32),
        ],
        scratch_shapes=[pltpu.VMEM((104, H), jnp.float32)],
    )(acc, h2p, dis, b2, gamma2, beta2, aw1, ab1, aw2,
      we, be, gammae, betae, wb, bb, gammab, betab)


@jax.jit
def kernel(x, edge_index, W1, b1, W2, b2, gamma1, beta1, gamma2, beta2,
           Aw1, Ab1, Aw2, Ab2, We, be, gammae, betae, Wb, bb, gammab, betab):
    pad = NW * EPP - E
    pad_src = jnp.arange(pad, dtype=edge_index.dtype) % N
    src_r = jnp.concatenate([edge_index[0], pad_src]).reshape(NW, EPP)
    sinks = SINK + jnp.arange(pad, dtype=edge_index.dtype) % (NPAD - N)
    dst_r = jnp.concatenate([edge_index[1], sinks]).reshape(NW, G, K)
    dst_flat = edge_index[1].reshape(NW, EPT)
    x_pad = jnp.pad(x, ((0, NPAD - N), (0, 0)))

    hist = _deg_kernel(dst_flat)
    h1p, dis = _pre_stage(x_pad, hist, W1)
    acc1 = _spmm_kernel(h1p, src_r, dst_r)
    h2p = _mid_stage(acc1, h1p, dis, b1, gamma1, beta1, W2)
    acc2 = _spmm_kernel(h2p, src_r, dst_r)
    emb, out = _tail_stage(acc2, h2p, dis, b2, gamma2, beta2, Aw1, Ab1, Aw2,
                           We, be, gammae, betae, Wb, bb, gammab, betab)
    return (out, emb)
